# baseline (speedup 1.0000x reference)
import functools

import jax
import jax.numpy as jnp
from jax import lax
from jax.experimental import pallas as pl
from jax.experimental.pallas import tpu as pltpu
from jax.experimental.pallas import tpu_sc as plsc

_B = 16384
_D = 64
_K = 5
_J = _K + 1
_N = 1000000
_NC = 2
_NS = 16
_NW = _NC * _NS
_BPW = _B // _NW
_C = 128
_NCH = _BPW // _C
_W32 = _D
_IDX_ROWS = _B * _J // _C

_GDN = lax.GatherDimensionNumbers(
    offset_dims=(), collapsed_slice_dims=(0,), start_index_map=(0,))
_LN2 = 0.6931471805599453


def _hsum_all_lanes(p, lanes):
    w = p
    for k in (8, 4, 2, 1):
        perm = jnp.bitwise_xor(lanes, k)
        w = w + lax.gather(w, perm[:, None], _GDN, (1,),
                           mode=lax.GatherScatterMode.PROMISE_IN_BOUNDS)
    return w


def _decode16(p16):
    m = jnp.bitwise_and(p16, 127) + 128
    e = jnp.bitwise_and(lax.shift_right_logical(p16, 7), 255)
    s = lax.shift_right_logical(p16, 15)
    sm = jnp.where(s == 1, -m, m).astype(jnp.float32)
    return sm * jnp.exp((e.astype(jnp.float32) - 134.0) * jnp.float32(_LN2))


def _decode_pair(w):
    lo = _decode16(jnp.bitwise_and(w, 65535))
    hi = _decode16(lax.shift_right_logical(w, 16))
    return lo, hi


@functools.partial(
    pl.kernel,
    out_type=(
        jax.ShapeDtypeStruct((_B,), jnp.float32),
        jax.ShapeDtypeStruct((_K * _B,), jnp.float32),
    ),
    mesh=plsc.VectorSubcoreMesh(core_axis_name="c", subcore_axis_name="s"),
    compiler_params=pltpu.CompilerParams(use_tc_tiling_on_sc=False),
    scratch_types=[
        pltpu.VMEM((_NCH, _C), jnp.int32),
        pltpu.VMEM((_NCH * _J, _C), jnp.int32),
        pltpu.VMEM((_C, _W32), jnp.int32),
        pltpu.VMEM((_J * _C, _W32), jnp.int32),
        pltpu.VMEM((_C,), jnp.float32),
        pltpu.VMEM((_K * _C,), jnp.float32),
        pltpu.SemaphoreType.DMA,
    ],
)
def _sc_scores(tab_hbm, center_hbm, idx2_hbm,
               pos_hbm, negs_hbm,
               cidx, ridx, crows, rows2, pos_s, negs_s, sem):
    wid = lax.axis_index("s") * _NC + lax.axis_index("c")
    lanes = lax.iota(jnp.int32, 16)

    for ci in range(_NCH):
        pltpu.sync_copy(center_hbm.at[pl.ds(wid * _BPW + ci * _C, _C)],
                        cidx.at[ci])
    for r8 in range(_NCH * _J // 8):
        pltpu.sync_copy(idx2_hbm.at[pl.ds(wid * (_NCH * _J) + r8 * 8, 8)],
                        ridx.at[pl.ds(r8 * 8, 8)])

    for ci in range(_NCH):
        base = wid * _BPW + ci * _C

        cps = [pltpu.async_copy(tab_hbm.at[cidx.at[ci]], crows, sem)]
        for r in range(_J):
            cps.append(pltpu.async_copy(
                tab_hbm.at[ridx.at[ci * _J + r]],
                rows2.at[pl.ds(r * _C, _C)], sem))
        for cp in cps:
            cp.wait()

        def group_body(g, carry):
            def rowstep(t, accs):
                i = g * 16 + t
                cvecs = []
                for h in range(2):
                    lo, hi = _decode_pair(crows[i, pl.ds(h * 16, 16)])
                    cvecs += [lo, hi]
                new = []
                for j in range(_J):
                    p = jnp.zeros((16,), jnp.float32)
                    for h in range(2):
                        lo, hi = _decode_pair(
                            rows2[i * _J + j, pl.ds(32 + h * 16, 16)])
                        p = p + cvecs[2 * h] * lo + cvecs[2 * h + 1] * hi
                    w = _hsum_all_lanes(p, lanes)
                    new.append(jnp.where(lanes == t, w, accs[j]))
                return tuple(new)

            zero = jnp.zeros((16,), jnp.float32)
            accs = lax.fori_loop(0, 16, rowstep, (zero,) * _J)
            pos_s[pl.ds(g * 16, 16)] = accs[0]
            for kk in range(_K):
                negs_s[pl.ds(kk * _C + g * 16, 16)] = accs[kk + 1]
            return carry

        lax.fori_loop(0, _C // 16, group_body, 0)

        pltpu.sync_copy(pos_s, pos_hbm.at[pl.ds(base, _C)])
        for kk in range(_K):
            pltpu.sync_copy(negs_s.at[pl.ds(kk * _C, _C)],
                            negs_hbm.at[pl.ds(kk * _B + base, _C)])


def _loss_body(pos_ref, negs_ref, out_ref):
    sp = jnp.sum(jax.nn.log_sigmoid(pos_ref[...]))
    sn = jnp.sum(jax.nn.log_sigmoid(-negs_ref[...]))
    out_ref[0, 0] = -(sp / _B) - (sn / (_B * _K))


_loss_call = pl.pallas_call(
    _loss_body,
    out_shape=jax.ShapeDtypeStruct((1, 1), jnp.float32),
    out_specs=pl.BlockSpec(memory_space=pltpu.SMEM),
)


def kernel(center, context, neg, embed_W, context_W):
    center = center.astype(jnp.int32)
    idx2 = jnp.concatenate(
        [context.astype(jnp.int32)[:, None], neg.astype(jnp.int32)], axis=1)
    idx2 = idx2.reshape(_IDX_ROWS, _C)
    packed = jnp.concatenate([embed_W.astype(jnp.bfloat16),
                              context_W.astype(jnp.bfloat16)], axis=1)
    tab = lax.bitcast_convert_type(packed.reshape(_N, _W32, 2), jnp.int32)
    pos, negs = _sc_scores(tab, center, idx2)
    loss = _loss_call(pos.reshape(_B // _C, _C),
                      negs.reshape(_K * _B // _C, _C))
    return loss[0, 0]


# speedup vs baseline: 2.1701x; 2.1701x over previous
import functools

import jax
import jax.numpy as jnp
from jax import lax
from jax.experimental import pallas as pl
from jax.experimental.pallas import tpu as pltpu
from jax.experimental.pallas import tpu_sc as plsc

_B = 16384
_D = 64
_K = 5
_J = _K + 1
_N = 1000000
_NC = 2
_NS = 16
_NW = _NC * _NS
_BPW = _B // _NW
_C = 128
_NCH = _BPW // _C
_W32 = _D
_IDX_ROWS = _B * _J // _C

_GDN = lax.GatherDimensionNumbers(
    offset_dims=(), collapsed_slice_dims=(0,), start_index_map=(0,))
_LN2 = 0.6931471805599453


def _hsum_all_lanes(p, lanes):
    w = p
    for k in (8, 4, 2, 1):
        perm = jnp.bitwise_xor(lanes, k)
        w = w + lax.gather(w, perm[:, None], _GDN, (1,),
                           mode=lax.GatherScatterMode.PROMISE_IN_BOUNDS)
    return w


def _decode16(p16):
    m = jnp.bitwise_and(p16, 127) + 128
    e = jnp.bitwise_and(lax.shift_right_logical(p16, 7), 255)
    s = lax.shift_right_logical(p16, 15)
    sm = jnp.where(s == 1, -m, m).astype(jnp.float32)
    return sm * jnp.exp((e.astype(jnp.float32) - 134.0) * jnp.float32(_LN2))


@functools.partial(
    pl.kernel,
    out_type=(
        jax.ShapeDtypeStruct((_B,), jnp.float32),
        jax.ShapeDtypeStruct((_K * _B,), jnp.float32),
    ),
    mesh=plsc.VectorSubcoreMesh(core_axis_name="c", subcore_axis_name="s"),
    compiler_params=pltpu.CompilerParams(use_tc_tiling_on_sc=False),
    scratch_types=[
        pltpu.VMEM((_NCH, _C), jnp.int32),
        pltpu.VMEM((_NCH * _J, _C), jnp.int32),
        pltpu.VMEM((_C, _W32), jnp.int32),
        pltpu.VMEM((_J * _C, _W32), jnp.int32),
        pltpu.VMEM((_C,), jnp.float32),
        pltpu.VMEM((_K * _C,), jnp.float32),
        pltpu.SemaphoreType.DMA,
    ],
)
def _sc_scores(tab_hbm, center_hbm, idx2_hbm,
               pos_hbm, negs_hbm,
               cidx, ridx, crows, rows2, pos_s, negs_s, sem):
    wid = lax.axis_index("s") * _NC + lax.axis_index("c")
    lanes = lax.iota(jnp.int32, 16)

    for ci in range(_NCH):
        pltpu.sync_copy(center_hbm.at[pl.ds(wid * _BPW + ci * _C, _C)],
                        cidx.at[ci])
    for r8 in range(_NCH * _J // 8):
        pltpu.sync_copy(idx2_hbm.at[pl.ds(wid * (_NCH * _J) + r8 * 8, 8)],
                        ridx.at[pl.ds(r8 * 8, 8)])

    for ci in range(_NCH):
        base = wid * _BPW + ci * _C

        cps = [pltpu.async_copy(tab_hbm.at[cidx.at[ci]], crows, sem)]
        for r in range(_J):
            cps.append(pltpu.async_copy(
                tab_hbm.at[ridx.at[ci * _J + r]],
                rows2.at[pl.ds(r * _C, _C)], sem))
        for cp in cps:
            cp.wait()

        def group_body(g, carry):
            def rowstep(t, accs):
                i = g * 16 + t
                cvecs = [
                    _decode16(jnp.bitwise_and(crows[i, pl.ds(h * 16, 16)],
                                              65535))
                    for h in range(4)]
                new = []
                for j in range(_J):
                    p = jnp.zeros((16,), jnp.float32)
                    for h in range(4):
                        xv = _decode16(lax.shift_right_logical(
                            rows2[i * _J + j, pl.ds(h * 16, 16)], 16))
                        p = p + cvecs[h] * xv
                    w = _hsum_all_lanes(p, lanes)
                    new.append(jnp.where(lanes == t, w, accs[j]))
                return tuple(new)

            zero = jnp.zeros((16,), jnp.float32)
            accs = lax.fori_loop(0, 16, rowstep, (zero,) * _J)
            pos_s[pl.ds(g * 16, 16)] = accs[0]
            for kk in range(_K):
                negs_s[pl.ds(kk * _C + g * 16, 16)] = accs[kk + 1]
            return carry

        lax.fori_loop(0, _C // 16, group_body, 0)

        pltpu.sync_copy(pos_s, pos_hbm.at[pl.ds(base, _C)])
        for kk in range(_K):
            pltpu.sync_copy(negs_s.at[pl.ds(kk * _C, _C)],
                            negs_hbm.at[pl.ds(kk * _B + base, _C)])


def _loss_body(pos_ref, negs_ref, out_ref):
    sp = jnp.sum(jax.nn.log_sigmoid(pos_ref[...]))
    sn = jnp.sum(jax.nn.log_sigmoid(-negs_ref[...]))
    out_ref[0, 0] = -(sp / _B) - (sn / (_B * _K))


_loss_call = pl.pallas_call(
    _loss_body,
    out_shape=jax.ShapeDtypeStruct((1, 1), jnp.float32),
    out_specs=pl.BlockSpec(memory_space=pltpu.SMEM),
)


def kernel(center, context, neg, embed_W, context_W):
    center = center.astype(jnp.int32)
    idx2 = jnp.concatenate(
        [context.astype(jnp.int32)[:, None], neg.astype(jnp.int32)], axis=1)
    idx2 = idx2.reshape(_IDX_ROWS, _C)
    eb = lax.bitcast_convert_type(embed_W, jnp.int32)
    cb = lax.bitcast_convert_type(context_W, jnp.int32)

    def _rne16(u):
        return lax.shift_right_logical(
            u + 32767 + jnp.bitwise_and(lax.shift_right_logical(u, 16), 1),
            16)

    tab = jnp.bitwise_or(_rne16(eb), lax.shift_left(_rne16(cb), 16))
    pos, negs = _sc_scores(tab, center, idx2)
    loss = _loss_call(pos.reshape(_B // _C, _C),
                      negs.reshape(_K * _B // _C, _C))
    return loss[0, 0]


# speedup vs baseline: 3.4589x; 1.5939x over previous
import functools

import jax
import jax.numpy as jnp
from jax import lax
from jax.experimental import pallas as pl
from jax.experimental.pallas import tpu as pltpu
from jax.experimental.pallas import tpu_sc as plsc

_B = 16384
_D = 64
_K = 5
_J = _K + 1
_N = 1000000
_NC = 2
_NS = 16
_NW = _NC * _NS
_BPW = _B // _NW
_C = 128
_NCH = _BPW // _C
_W32 = _D
_IDX_ROWS = _B * _J // _C

_GDN = lax.GatherDimensionNumbers(
    offset_dims=(), collapsed_slice_dims=(0,), start_index_map=(0,))
_LN2 = 0.6931471805599453


def _hsum_all_lanes(p, lanes):
    w = p
    for k in (8, 4, 2, 1):
        perm = jnp.bitwise_xor(lanes, k)
        w = w + lax.gather(w, perm[:, None], _GDN, (1,),
                           mode=lax.GatherScatterMode.PROMISE_IN_BOUNDS)
    return w


def _decode16(p16):
    m = jnp.bitwise_and(p16, 127) + 128
    e = jnp.bitwise_and(lax.shift_right_logical(p16, 7), 255)
    s = lax.shift_right_logical(p16, 15)
    sm = jnp.where(s == 1, -m, m).astype(jnp.float32)
    return sm * jnp.exp((e.astype(jnp.float32) - 134.0) * jnp.float32(_LN2))


@functools.partial(
    pl.kernel,
    out_type=(
        jax.ShapeDtypeStruct((_B,), jnp.float32),
        jax.ShapeDtypeStruct((_K * _B,), jnp.float32),
    ),
    mesh=plsc.VectorSubcoreMesh(core_axis_name="c", subcore_axis_name="s"),
    compiler_params=pltpu.CompilerParams(use_tc_tiling_on_sc=True),
    scratch_types=[
        pltpu.VMEM((_NCH, _C), jnp.int32),
        pltpu.VMEM((_NCH * _J, _C), jnp.int32),
        pltpu.VMEM((_C, 128), jnp.float32),
        pltpu.VMEM((_J * _C, 128), jnp.float32),
        pltpu.VMEM((_C,), jnp.float32),
        pltpu.VMEM((_K * _C,), jnp.float32),
        pltpu.SemaphoreType.DMA,
    ],
)
def _sc_scores(tab_hbm, center_hbm, idx2_hbm,
               pos_hbm, negs_hbm,
               cidx, ridx, crows, rows2, pos_s, negs_s, sem):
    wid = lax.axis_index("s") * _NC + lax.axis_index("c")
    lanes = lax.iota(jnp.int32, 16)

    for ci in range(_NCH):
        pltpu.sync_copy(center_hbm.at[pl.ds(wid * _BPW + ci * _C, _C)],
                        cidx.at[ci])
    for r8 in range(_NCH * _J // 8):
        pltpu.sync_copy(idx2_hbm.at[pl.ds(wid * (_NCH * _J) + r8 * 8, 8)],
                        ridx.at[pl.ds(r8 * 8, 8)])

    for ci in range(_NCH):
        base = wid * _BPW + ci * _C

        cps = [pltpu.async_copy(tab_hbm.at[cidx.at[ci]], crows, sem)]
        for r in range(_J):
            cps.append(pltpu.async_copy(
                tab_hbm.at[ridx.at[ci * _J + r]],
                rows2.at[pl.ds(r * _C, _C)], sem))
        for cp in cps:
            cp.wait()

        def group_body(g, carry):
            def rowstep(t, accs):
                i = g * 16 + t
                cvecs = [crows[i, pl.ds(h * 16, 16)] for h in range(4)]
                new = []
                for j in range(_J):
                    p = jnp.zeros((16,), jnp.float32)
                    for h in range(4):
                        xv = rows2[i * _J + j, pl.ds(64 + h * 16, 16)]
                        p = p + cvecs[h] * xv
                    w = _hsum_all_lanes(p, lanes)
                    new.append(jnp.where(lanes == t, w, accs[j]))
                return tuple(new)

            zero = jnp.zeros((16,), jnp.float32)
            accs = lax.fori_loop(0, 16, rowstep, (zero,) * _J)
            pos_s[pl.ds(g * 16, 16)] = accs[0]
            for kk in range(_K):
                negs_s[pl.ds(kk * _C + g * 16, 16)] = accs[kk + 1]
            return carry

        lax.fori_loop(0, _C // 16, group_body, 0)

        pltpu.sync_copy(pos_s, pos_hbm.at[pl.ds(base, _C)])
        for kk in range(_K):
            pltpu.sync_copy(negs_s.at[pl.ds(kk * _C, _C)],
                            negs_hbm.at[pl.ds(kk * _B + base, _C)])


def _loss_body(pos_ref, negs_ref, out_ref):
    sp = jnp.sum(jax.nn.log_sigmoid(pos_ref[...]))
    sn = jnp.sum(jax.nn.log_sigmoid(-negs_ref[...]))
    out_ref[0, 0] = -(sp / _B) - (sn / (_B * _K))


_loss_call = pl.pallas_call(
    _loss_body,
    out_shape=jax.ShapeDtypeStruct((1, 1), jnp.float32),
    out_specs=pl.BlockSpec(memory_space=pltpu.SMEM),
)


def kernel(center, context, neg, embed_W, context_W):
    center = center.astype(jnp.int32)
    idx2 = jnp.concatenate(
        [context.astype(jnp.int32)[:, None], neg.astype(jnp.int32)], axis=1)
    idx2 = idx2.reshape(_IDX_ROWS, _C)
    tab = jnp.concatenate([embed_W, context_W], axis=1)
    pos, negs = _sc_scores(tab, center, idx2)
    loss = _loss_call(pos.reshape(_B // _C, _C),
                      negs.reshape(_K * _B // _C, _C))
    return loss[0, 0]


# speedup vs baseline: 3.4626x; 1.0011x over previous
import functools

import jax
import jax.numpy as jnp
from jax import lax
from jax.experimental import pallas as pl
from jax.experimental.pallas import tpu as pltpu
from jax.experimental.pallas import tpu_sc as plsc

_B = 16384
_D = 64
_K = 5
_J = _K + 1
_N = 1000000
_NC = 2
_NS = 16
_NW = _NC * _NS
_BPW = _B // _NW
_C = 128
_NCH = _BPW // _C
_IDX_ROWS = _B * _J // _C

_GDN = lax.GatherDimensionNumbers(
    offset_dims=(), collapsed_slice_dims=(0,), start_index_map=(0,))


def _hsum_all_lanes(p, lanes):
    w = p
    for k in (8, 4, 2, 1):
        perm = jnp.bitwise_xor(lanes, k)
        w = w + lax.gather(w, perm[:, None], _GDN, (1,),
                           mode=lax.GatherScatterMode.PROMISE_IN_BOUNDS)
    return w


@functools.partial(
    pl.kernel,
    out_type=(
        jax.ShapeDtypeStruct((_B,), jnp.float32),
        jax.ShapeDtypeStruct((_K * _B,), jnp.float32),
    ),
    mesh=plsc.VectorSubcoreMesh(core_axis_name="c", subcore_axis_name="s"),
    compiler_params=pltpu.CompilerParams(use_tc_tiling_on_sc=True),
    scratch_types=[
        pltpu.VMEM((_NCH, _C), jnp.int32),
        pltpu.VMEM((_NCH * _J, _C), jnp.int32),
        pltpu.VMEM((_C, 128), jnp.float32),
        pltpu.VMEM((_J * _C, 128), jnp.float32),
        pltpu.VMEM((_C,), jnp.float32),
        pltpu.VMEM((_K * _C,), jnp.float32),
        pltpu.SemaphoreType.DMA,
    ],
)
def _sc_scores(tab_hbm, center_hbm, idx2_hbm,
               pos_hbm, negs_hbm,
               cidx, ridx, crows, rows2, pos_s, negs_s, sem):
    wid = lax.axis_index("s") * _NC + lax.axis_index("c")
    lanes = lax.iota(jnp.int32, 16)

    for ci in range(_NCH):
        pltpu.sync_copy(center_hbm.at[pl.ds(wid * _BPW + ci * _C, _C)],
                        cidx.at[ci])
    for r8 in range(_NCH * _J // 8):
        pltpu.sync_copy(idx2_hbm.at[pl.ds(wid * (_NCH * _J) + r8 * 8, 8)],
                        ridx.at[pl.ds(r8 * 8, 8)])

    for ci in range(_NCH):
        base = wid * _BPW + ci * _C

        cps = [pltpu.async_copy(tab_hbm.at[cidx.at[ci]], crows, sem)]
        for r in range(_J):
            cps.append(pltpu.async_copy(
                tab_hbm.at[ridx.at[ci * _J + r]],
                rows2.at[pl.ds(r * _C, _C)], sem))
        for cp in cps:
            cp.wait()

        def group_body(g, carry):
            def rowstep(t, accs):
                i = g * 16 + t
                cvecs = [crows[i, pl.ds(h * 16, 16)] for h in range(4)]
                new = []
                for j in range(_J):
                    p = jnp.zeros((16,), jnp.float32)
                    for h in range(4):
                        xv = rows2[i * _J + j, pl.ds(64 + h * 16, 16)]
                        p = p + cvecs[h] * xv
                    w = _hsum_all_lanes(p, lanes)
                    new.append(jnp.where(lanes == t, w, accs[j]))
                return tuple(new)

            zero = jnp.zeros((16,), jnp.float32)
            accs = lax.fori_loop(0, 16, rowstep, (zero,) * _J)
            pos_s[pl.ds(g * 16, 16)] = accs[0]
            for kk in range(_K):
                negs_s[pl.ds(kk * _C + g * 16, 16)] = accs[kk + 1]
            return carry

        lax.fori_loop(0, _C // 16, group_body, 0)

        pltpu.sync_copy(pos_s, pos_hbm.at[pl.ds(base, _C)])
        for kk in range(_K):
            pltpu.sync_copy(negs_s.at[pl.ds(kk * _C, _C)],
                            negs_hbm.at[pl.ds(kk * _B + base, _C)])


def _loss_body(pos_ref, negs_ref, out_ref):
    sp = jnp.sum(jax.nn.log_sigmoid(pos_ref[...]))
    sn = jnp.sum(jax.nn.log_sigmoid(-negs_ref[...]))
    out_ref[0, 0] = -(sp / _B) - (sn / (_B * _K))


_loss_call = pl.pallas_call(
    _loss_body,
    out_shape=jax.ShapeDtypeStruct((1, 1), jnp.float32),
    out_specs=pl.BlockSpec(memory_space=pltpu.SMEM),
)


def kernel(center, context, neg, embed_W, context_W):
    center = center.astype(jnp.int32)
    idx2 = jnp.concatenate(
        [context.astype(jnp.int32)[:, None], neg.astype(jnp.int32)], axis=1)
    idx2 = idx2.reshape(_IDX_ROWS, _C)
    tab = jnp.concatenate([embed_W, context_W], axis=1)
    pos, negs = _sc_scores(tab, center, idx2)
    loss = _loss_call(pos.reshape(_B // _C, _C),
                      negs.reshape(_K * _B // _C, _C))
    return loss[0, 0]
